# Initial kernel scaffold; baseline (speedup 1.0000x reference)
import jax
import jax.numpy as jnp
from jax.experimental import pallas as pl


def kernel(x, pos, edge_index, params):
    raise NotImplementedError("write your pallas kernel here")


# speedup vs baseline: 24.2538x; 24.2538x over previous
import functools

import jax
import jax.numpy as jnp
import numpy as np
from jax import lax
from jax.experimental import pallas as pl
from jax.experimental.pallas import tpu as pltpu
from jax.experimental.pallas import tpu_sc as plsc

N = 10000
E = 320000
TW = 25
HID = 128
NLAYER = 6
PDE_L = 16.0
PDE_TMAX = 4.0
PDE_DT = 0.01

NP = 10240
BN = 1024
GRID = NP // BN
NCORE = 2
NSUB = 16
CHUNK = 128
NCHUNK = 157
EPT = NCHUNK * CHUNK
EP = EPT * NSUB
RSTRIP = NP // NSUB


def _swish(v):
    return v * jax.nn.sigmoid(v)


def _encoder_body(x_ref, pos_ref, wi_ref, whc_ref, why_ref, bi_ref,
                  w1_ref, b1_ref, w2_ref, b2_ref, vmat_ref,
                  h_ref, q_ref):
    xb = x_ref[:, :]
    posx = pos_ref[:, 1:2] * (1.0 / PDE_L)
    post = pos_ref[:, 0:1] * (1.0 / PDE_TMAX)
    wi = wi_ref[:, :]
    whc = whc_ref[:, :]
    why = why_ref[:, :]
    stat = posx * wi[0:1, :] + post * wi[3:4, :] + bi_ref[:, :]
    y = jnp.zeros((BN, HID), jnp.float32)
    z = jnp.zeros((BN, HID), jnp.float32)
    for t in range(TW):
        uin = (stat + xb[:, t:t + 1] * wi[1:2, :]
               + xb[:, TW + t:TW + t + 1] * wi[2:3, :]
               + (PDE_DT * (t + 1)) * wi[3:4, :])
        hy = jnp.dot(y, whc, preferred_element_type=jnp.float32)
        g1 = jax.nn.sigmoid(uin[:, 0:128] + hy[:, 0:128])
        g2 = jax.nn.sigmoid(uin[:, 128:256] + hy[:, 128:256])
        z = (1.0 - g1) * z + g1 * jnp.tanh(uin[:, 256:384] + hy[:, 256:384])
        y = (1.0 - g2) * y + g2 * jnp.tanh(
            uin[:, 384:512] + jnp.dot(z, why, preferred_element_type=jnp.float32))
    h = y @ w1_ref[:, :] + b1_ref[:, :]
    h = _swish(h)
    h = h @ w2_ref[:, :] + b2_ref[:, :]
    h = _swish(h)
    h_ref[:, :] = h
    q_ref[:, :] = (jnp.dot(xb, vmat_ref[0:2 * TW, :],
                           preferred_element_type=jnp.float32)
                   + posx * vmat_ref[2 * TW:2 * TW + 1, :])


def _pre_body(li, h_ref, q_ref, wcat_ref, hxg_ref, hxm_ref, scal_ref, bm_ref):
    p = jnp.dot(h_ref[:, :], wcat_ref[:, :], preferred_element_type=jnp.float32)
    qg = q_ref[:, li:li + 1]
    qm = q_ref[:, NLAYER + li:NLAYER + li + 1]
    ag = p[:, 256:257] + qg
    bg = p[:, 257:258] - qg
    am = p[:, 258:259] + qm
    bm = p[:, 259:260] - qm
    hxg_ref[:, :] = p[:, 0:128]
    hxm_ref[:, :] = p[:, 128:256]
    scal_ref[:, :] = jnp.concatenate(
        [ag, bg, am, bm, jnp.zeros((BN, 4), jnp.float32)], axis=1)
    i8 = lax.broadcasted_iota(jnp.int32, (1, 1, 8), 2)
    mg = jnp.max(ag)
    mm = jnp.max(am)
    bm_ref[:, :, :] = jnp.where(i8 == 0, mg, jnp.where(i8 == 1, mm, 0.0))


def _agg_body(hxg, hxm, ab, mx, srcr, dstr, out, outden,
              si_v, di_v, a_v, b_v, mx_v, g_v, ex_v, den_v,
              acc_sh, sem):
    c = lax.axis_index("c")
    s = lax.axis_index("s")
    pltpu.sync_copy(ab.at[c, 0], a_v)
    pltpu.sync_copy(ab.at[c, 1], b_v)
    pltpu.sync_copy(mx.at[c], mx_v)

    zvec = jnp.zeros((16,), jnp.float32)

    def zrow(r, _):
        for j in range(HID // 16):
            g_v[r, pl.ds(16 * j, 16)] = zvec
        return 0

    lax.fori_loop(0, CHUNK, zrow, 0)

    def zden(r, _):
        den_v[pl.ds(16 * r, 16)] = zvec
        return 0

    lax.fori_loop(0, NP // 16, zden, 0)

    def zacc(j, _):
        pltpu.sync_copy(g_v, acc_sh.at[pl.ds(RSTRIP * s + CHUNK * j, CHUNK)])
        return 0

    lax.fori_loop(0, RSTRIP // CHUNK, zacc, 0)
    plsc.subcore_barrier()

    iota16 = lax.iota(jnp.int32, 16)
    mxv = mx_v[:]

    def chunk_body(j, _):
        pltpu.sync_copy(srcr.at[s, j], si_v.at[0])
        pltpu.sync_copy(dstr.at[s, j], di_v.at[0])

        @pl.when(c == 0)
        def _():
            pltpu.async_copy(hxg.at[si_v.at[0]], g_v, sem)

        @pl.when(c == 1)
        def _():
            pltpu.async_copy(hxm.at[si_v.at[0]], g_v, sem)

        def kgrp(k, _):
            si = si_v[0, pl.ds(16 * k, 16)]
            di = di_v[0, pl.ds(16 * k, 16)]
            a = plsc.load_gather(a_v, [si])
            b = plsc.load_gather(b_v, [di])
            t = a + b
            al = jnp.maximum(t, 0.2 * t)
            cb = b + mxv
            cc = jnp.maximum(cb, 0.2 * cb)
            exv = jnp.exp(al - cc)
            ex_v[pl.ds(16 * k, 16)] = exv

            def lanes(l, _):
                plsc.addupdate_scatter(den_v, [di], exv, mask=iota16 == l)
                return 0

            lax.fori_loop(0, 16, lanes, 0)
            return 0

        lax.fori_loop(0, CHUNK // 16, kgrp, 0)

        pltpu.make_async_copy(hxg.at[si_v.at[0]], g_v, sem).wait()

        def rowfn(r, _):
            e = plsc.load_gather(ex_v, [jnp.full((16,), r, jnp.int32)])
            for jj in range(HID // 16):
                g_v[r, pl.ds(16 * jj, 16)] = g_v[r, pl.ds(16 * jj, 16)] * e
            return 0

        lax.fori_loop(0, CHUNK, rowfn, 0)

        pltpu.sync_copy(g_v, acc_sh.at[di_v.at[0]], add=True)
        return 0

    lax.fori_loop(0, NCHUNK, chunk_body, 0)
    pltpu.sync_copy(den_v, outden.at[c, s])
    plsc.subcore_barrier()
    pltpu.sync_copy(acc_sh.at[pl.ds(RSTRIP * s, RSTRIP)],
                    out.at[c, pl.ds(RSTRIP * s, RSTRIP)])


def _combine_body(h_ref, agg_ref, den_ref, bg_ref, bm_ref, o_ref):
    accg = agg_ref[0, :, :]
    accm = agg_ref[1, :, :]
    deng = jnp.sum(den_ref[0, :, :], axis=0)[:, None]
    denm = jnp.sum(den_ref[1, :, :], axis=0)[:, None]
    og = accg / (deng + 1e-16) + bg_ref[:, :]
    om = accm / (denm + 1e-16) + bm_ref[:, :]
    tau = jax.nn.sigmoid(og)
    o_ref[:, :] = (1.0 - tau) * h_ref[:, :] + tau * _swish(om)


def _decoder_body(x_ref, h_ref, wd_ref, bd_ref, w1_ref, b1_ref,
                  w2_ref, b2_ref, dtv_ref, o_ref):
    h2 = _swish(jnp.dot(h_ref[:, :], wd_ref[:, :],
                        preferred_element_type=jnp.float32) + bd_ref[:, :])
    y1 = _swish(jnp.dot(h2, w1_ref[:, :],
                        preferred_element_type=jnp.float32) + b1_ref[:, :])
    diff = jnp.dot(y1, w2_ref[:, :],
                   preferred_element_type=jnp.float32) + b2_ref[:, :]
    o_ref[:, :] = x_ref[:, :] + dtv_ref[:, :] * diff


def _full(shape):
    nd = len(shape)
    return pl.BlockSpec(shape, lambda i, _n=nd: (0,) * _n)


def _rows(width):
    return pl.BlockSpec((BN, width), lambda i: (i, 0))


def kernel(x, pos, edge_index, params):
    f32 = jnp.float32
    lem = params['lem']

    wi_cat = jnp.concatenate([lem['Wi1'], lem['Wi2'], lem['Wiz'], lem['Wiy']], axis=1)
    whc = jnp.concatenate([lem['Wh1'], lem['Wh2'], lem['Whz']], axis=1)
    bi = jnp.concatenate([lem['b1'], lem['b2'], lem['bz'], lem['by']]).reshape(1, 512)
    b1 = params['mlp_b1'].reshape(1, HID)
    b2 = params['mlp_b2'].reshape(1, HID)

    vcols = []
    for nm in ('gate', 'main'):
        for i in range(NLAYER):
            p = params[nm][i]
            vcols.append(p['lin_edge_w'] @ p['att_edge'])
    vcols += [jnp.zeros((2 * TW + 1,), f32)] * 4
    vmat = jnp.stack(vcols, axis=1)

    wcats, biases = [], []
    for i in range(NLAYER):
        pg, pm = params['gate'][i], params['main'][i]
        wcat = jnp.concatenate([
            pg['lin_w'], pm['lin_w'],
            (pg['lin_w'] @ pg['att_src'])[:, None],
            (pg['lin_w'] @ pg['att_dst'])[:, None],
            (pm['lin_w'] @ pm['att_src'])[:, None],
            (pm['lin_w'] @ pm['att_dst'])[:, None],
            jnp.zeros((HID, 124), f32)], axis=1)
        wcats.append(wcat)
        biases.append((pg['bias'].reshape(1, HID), pm['bias'].reshape(1, HID)))

    o_, i_, p_, k_ = np.indices((8, 2, 38, 16))
    w1d = jnp.zeros((2 * HID, 8 * 38), f32).at[
        (i_ * HID + 3 * p_ + k_).ravel(), (o_ * 38 + p_).ravel()].set(
        params['conv1_w'][o_.ravel(), i_.ravel(), k_.ravel()])
    o_, i_, p_, k_ = np.indices((2, 8, 25, 14))
    w2d = jnp.zeros((8 * 38, 2 * TW), f32).at[
        (i_ * 38 + p_ + k_).ravel(), (o_ * TW + p_).ravel()].set(
        params['conv2_w'][o_.ravel(), i_.ravel(), k_.ravel()])
    b1c = jnp.repeat(params['conv1_b'], 38).reshape(1, 8 * 38)
    b2c = jnp.repeat(params['conv2_b'], TW).reshape(1, 2 * TW)
    dtv = jnp.asarray(np.tile(PDE_DT * (np.arange(TW) + 1.0), 2),
                      f32).reshape(1, 2 * TW)

    xp = jnp.pad(x, ((0, NP - N), (0, 0)))
    posp = jnp.pad(pos, ((0, NP - N), (0, 0)))
    epad = jnp.full((2, EP - E), NP - 1, jnp.int32)
    ei = jnp.concatenate([edge_index, epad], axis=1)
    src3 = ei[0].reshape(NSUB, NCHUNK, CHUNK)
    dst3 = ei[1].reshape(NSUB, NCHUNK, CHUNK)

    h, q = pl.pallas_call(
        _encoder_body,
        grid=(GRID,),
        in_specs=[_rows(2 * TW), _rows(2), _full((4, 512)), _full((HID, 384)),
                  _full((HID, HID)), _full((1, 512)), _full((HID, HID)),
                  _full((1, HID)), _full((HID, HID)), _full((1, HID)),
                  _full((2 * TW + 1, 16))],
        out_specs=[_rows(HID), _rows(16)],
        out_shape=[jax.ShapeDtypeStruct((NP, HID), f32),
                   jax.ShapeDtypeStruct((NP, 16), f32)],
    )(xp, posp, wi_cat, whc, lem['Why'], bi, params['mlp_w1'], b1,
      params['mlp_w2'], b2, vmat)

    mesh = plsc.VectorSubcoreMesh(core_axis_name="c", subcore_axis_name="s",
                                  num_cores=NCORE, num_subcores=NSUB)
    agg_call = pl.kernel(
        _agg_body,
        out_type=[jax.ShapeDtypeStruct((NCORE, NP, HID), f32),
                  jax.ShapeDtypeStruct((NCORE, NSUB, NP), f32)],
        mesh=mesh,
        compiler_params=pltpu.CompilerParams(needs_layout_passes=False),
        scratch_types=[
            pltpu.VMEM((1, CHUNK), jnp.int32),
            pltpu.VMEM((1, CHUNK), jnp.int32),
            pltpu.VMEM((NP,), f32),
            pltpu.VMEM((NP,), f32),
            pltpu.VMEM((16,), f32),
            pltpu.VMEM((CHUNK, HID), f32),
            pltpu.VMEM((CHUNK,), f32),
            pltpu.VMEM((NP,), f32),
            pltpu.VMEM_SHARED((NP, HID), f32),
            pltpu.SemaphoreType.DMA,
        ],
    )

    for i in range(NLAYER):
        hxg, hxm, scal, bmx = pl.pallas_call(
            functools.partial(_pre_body, i),
            grid=(GRID,),
            in_specs=[_rows(HID), _rows(16), _full((HID, 384))],
            out_specs=[_rows(HID), _rows(HID), _rows(8),
                       pl.BlockSpec((1, 1, 8), lambda g: (g, 0, 0))],
            out_shape=[jax.ShapeDtypeStruct((NP, HID), f32),
                       jax.ShapeDtypeStruct((NP, HID), f32),
                       jax.ShapeDtypeStruct((NP, 8), f32),
                       jax.ShapeDtypeStruct((GRID, 1, 8), f32)],
        )(h, q, wcats[i])

        ab = scal[:, :4].T.reshape(2, 2, NP)
        mg = jnp.max(bmx[:, 0, 0])
        mm = jnp.max(bmx[:, 0, 1])
        mx = jnp.stack([jnp.full((16,), mg, f32), jnp.full((16,), mm, f32)])

        agg, den = agg_call(hxg, hxm, ab, mx, src3, dst3)

        h = pl.pallas_call(
            _combine_body,
            grid=(GRID,),
            in_specs=[_rows(HID),
                      pl.BlockSpec((2, BN, HID), lambda g: (0, g, 0)),
                      pl.BlockSpec((2, NSUB, BN), lambda g: (0, 0, g)),
                      _full((1, HID)), _full((1, HID))],
            out_specs=_rows(HID),
            out_shape=jax.ShapeDtypeStruct((NP, HID), f32),
        )(h, agg, den, biases[i][0], biases[i][1])

    out = pl.pallas_call(
        _decoder_body,
        grid=(GRID,),
        in_specs=[_rows(2 * TW), _rows(HID), _full((HID, 2 * HID)),
                  _full((1, 2 * HID)), _full((2 * HID, 8 * 38)),
                  _full((1, 8 * 38)), _full((8 * 38, 2 * TW)),
                  _full((1, 2 * TW)), _full((1, 2 * TW))],
        out_specs=_rows(2 * TW),
        out_shape=jax.ShapeDtypeStruct((NP, 2 * TW), f32),
    )(xp, h, params['double_w'], params['double_b'].reshape(1, 2 * HID),
      w1d, b1c, w2d, b2c, dtv)
    return out[:N]
